# Initial kernel scaffold; baseline (speedup 1.0000x reference)
import jax
import jax.numpy as jnp
from jax.experimental import pallas as pl


def kernel(lateral_inflow, log_manning_n, length, slope, downstream_idx):
    raise NotImplementedError("write your pallas kernel here")


# speedup vs baseline: 9.3797x; 9.3797x over previous
import functools

import jax
import jax.numpy as jnp
from jax import lax
from jax.experimental import pallas as pl
from jax.experimental.pallas import tpu as pltpu
from jax.experimental.pallas import tpu_sc as plsc

N = 50000
T = 64
DT = 86400.0
NSUB = 4
OUTLET = N - 1

NS = 16
LANES = 16
CHUNK = 3200
NPAD = NS * CHUNK
NDUMP = 8
IDXW = 128
KROWS = CHUNK // IDXW
NVREG = CHUNK // LANES
DT_SUB = DT / NSUB

OUT_SUBCORE = OUTLET // CHUNK
OUT_LOCAL = OUTLET - OUT_SUBCORE * CHUNK
OUT_VREG = OUT_LOCAL // LANES
OUT_LANE = OUT_LOCAL % LANES

_LN2_HI = 0.693359375
_LN2_LO = -2.12194440e-4
_SQRT2 = 1.41421356237
_C27 = 0.27 ** (2.0 / 3.0)


def _ln16(x):
    bits = plsc.bitcast(x, jnp.int32)
    e = lax.shift_right_logical(bits, 23) - 127
    m = plsc.bitcast(
        jnp.bitwise_or(jnp.bitwise_and(bits, 0x007FFFFF), 0x3F800000),
        jnp.float32,
    )
    big = m > _SQRT2
    m = jnp.where(big, m * 0.5, m)
    e = jnp.where(big, e + 1, e)
    ef = e.astype(jnp.float32)
    f = m - 1.0
    z = f * f
    y = jnp.float32(7.0376836292e-2)
    y = y * f - 1.1514610310e-1
    y = y * f + 1.1676998740e-1
    y = y * f - 1.2420140846e-1
    y = y * f + 1.4249322787e-1
    y = y * f - 1.6668057665e-1
    y = y * f + 2.0000714765e-1
    y = y * f - 2.4999993993e-1
    y = y * f + 3.3333331174e-1
    y = f * z * y
    y = y + ef * _LN2_LO
    y = y - 0.5 * z
    return f + y + ef * _LN2_HI


_mesh = plsc.VectorSubcoreMesh(core_axis_name="c", subcore_axis_name="s")


@functools.partial(
    pl.kernel,
    out_type=jax.ShapeDtypeStruct((T * LANES,), jnp.float32),
    mesh=_mesh,
    compiler_params=pltpu.CompilerParams(needs_layout_passes=False),
    scratch_types=[
        pltpu.VMEM_SHARED((NPAD + NDUMP,), jnp.float32),
        pltpu.VMEM((CHUNK,), jnp.float32),
        pltpu.VMEM((CHUNK,), jnp.float32),
        pltpu.VMEM((CHUNK,), jnp.float32),
        pltpu.VMEM((CHUNK,), jnp.float32),
        pltpu.VMEM((CHUNK,), jnp.float32),
        pltpu.VMEM((CHUNK,), jnp.float32),
        pltpu.VMEM((CHUNK,), jnp.float32),
        pltpu.VMEM((CHUNK,), jnp.float32),
        pltpu.VMEM((KROWS, IDXW), jnp.int32),
        pltpu.VMEM((T * LANES,), jnp.float32),
    ],
)
def _routing_kernel(
    lat_hbm, logn_hbm, len_hbm, slope_hbm, dst_hbm, out_hbm,
    shared_up, q_v, in_v, up_v, lat_v, celc_v, xc_v, len_v, zero_v,
    idx_v, out_v,
):
    cid = lax.axis_index("c")
    sid = lax.axis_index("s")
    base = sid * CHUNK

    pltpu.sync_copy(dst_hbm.at[sid], idx_v)
    pltpu.sync_copy(len_hbm.at[pl.ds(base, CHUNK)], len_v)
    pltpu.sync_copy(logn_hbm.at[pl.ds(base, CHUNK)], up_v)
    pltpu.sync_copy(slope_hbm.at[pl.ds(base, CHUNK)], lat_v)

    def init_body(j, _):
        ds = pl.ds(j * LANES, LANES)
        ln_n = up_v[ds]
        sl = lat_v[ds]
        ln = len_v[ds]
        celc_v[ds] = (5.0 / 3.0) * _C27 * jnp.exp(0.5 * _ln16(sl) - ln_n)
        xc_v[ds] = 1.0 / (14.4 * sl * ln)
        ones = jnp.full((LANES,), 1.0, jnp.float32)
        q_v[ds] = ones
        in_v[ds] = ones
        zero_v[ds] = jnp.zeros((LANES,), jnp.float32)
        return 0

    lax.fori_loop(0, NVREG, init_body, 0)

    def substep(s, _):
        pltpu.sync_copy(zero_v, shared_up.at[pl.ds(base, CHUNK)])
        plsc.subcore_barrier()

        def scat_body(k, _):
            pltpu.sync_copy(
                q_v.at[pl.ds(k * IDXW, IDXW)],
                shared_up.at[idx_v.at[k]],
                add=True,
            )
            return 0

        lax.fori_loop(0, KROWS, scat_body, 0)
        plsc.subcore_barrier()

        pltpu.sync_copy(shared_up.at[pl.ds(base, CHUNK)], up_v)
        plsc.subcore_barrier()

        def mc_body(j, _):
            ds = pl.ds(j * LANES, LANES)
            q = q_v[ds]
            inflow = lat_v[ds] + up_v[ds]
            qref = jnp.maximum(0.5 * (inflow + q), 1e-6)
            lnq = _ln16(qref)
            sqrtq = jnp.exp(0.5 * lnq)
            p02 = jnp.exp(0.2 * lnq)
            cel = jnp.maximum(celc_v[ds] * p02, 1e-4)
            rcel = 1.0 / cel
            k2 = 2.0 * len_v[ds] * rcel
            x = jnp.clip(0.5 - xc_v[ds] * sqrtq * rcel, 0.0, 0.5)
            a = k2 * x
            b = k2 - a
            rden = 1.0 / (b + DT_SUB)
            c0 = (DT_SUB - a) * rden
            c1 = (DT_SUB + a) * rden
            c2 = (b - DT_SUB) * rden
            q_v[ds] = jnp.maximum(c0 * inflow + c1 * in_v[ds] + c2 * q, 0.0)
            in_v[ds] = inflow
            return 0

        lax.fori_loop(0, NVREG, mc_body, 0)
        return 0

    def timestep(t, _):
        pltpu.sync_copy(lat_hbm.at[t, pl.ds(base, CHUNK)], lat_v)
        lax.fori_loop(0, NSUB, substep, 0)
        out_v[pl.ds(t * LANES, LANES)] = q_v[pl.ds(OUT_VREG * LANES, LANES)]
        return 0

    lax.fori_loop(0, T, timestep, 0)

    @pl.when(jnp.logical_and(cid == 0, sid == OUT_SUBCORE))
    def _():
        pltpu.sync_copy(out_v, out_hbm)


def kernel(lateral_inflow, log_manning_n, length, slope, downstream_idx):
    pad = NPAD - N
    lat = jnp.pad(lateral_inflow, ((0, 0), (0, pad)))
    logn = jnp.pad(log_manning_n, (0, pad))
    leng = jnp.pad(length, (0, pad), constant_values=1000.0)
    slp = jnp.pad(slope, (0, pad), constant_values=0.01)
    pad_idx = NPAD + (jnp.arange(pad, dtype=jnp.int32) % NDUMP)
    dst = jnp.concatenate([downstream_idx.astype(jnp.int32), pad_idx])
    dst = dst.reshape(NS, KROWS, IDXW)
    out = _routing_kernel(lat, logn, leng, slp, dst)
    return out.reshape(T, LANES)[:, OUT_LANE]


# speedup vs baseline: 35.1443x; 3.7468x over previous
import functools

import jax
import jax.numpy as jnp
from jax import lax
from jax.experimental import pallas as pl
from jax.experimental.pallas import tpu as pltpu
from jax.experimental.pallas import tpu_sc as plsc

N = 50000
T = 64
DT = 86400.0
NSUB = 4
OUTLET = N - 1

NS = 16
LANES = 16
CHUNK = 3200
NPAD = NS * CHUNK
NDUMP = 8
IDXW = 128
KROWS = CHUNK // IDXW
NVREG = CHUNK // LANES
DT_SUB = DT / NSUB

OUT_SUBCORE = OUTLET // CHUNK
OUT_LOCAL = OUTLET - OUT_SUBCORE * CHUNK
OUT_VREG = OUT_LOCAL // LANES
OUT_LANE = OUT_LOCAL % LANES

_LN2_HI = 0.693359375
_LN2_LO = -2.12194440e-4
_SQRT2 = 1.41421356237
_C27 = 0.27 ** (2.0 / 3.0)


def _ln16(x):
    bits = plsc.bitcast(x, jnp.int32)
    e = lax.shift_right_logical(bits, 23) - 127
    m = plsc.bitcast(
        jnp.bitwise_or(jnp.bitwise_and(bits, 0x007FFFFF), 0x3F800000),
        jnp.float32,
    )
    big = m > _SQRT2
    m = jnp.where(big, m * 0.5, m)
    e = jnp.where(big, e + 1, e)
    ef = e.astype(jnp.float32)
    f = m - 1.0
    z = f * f
    y = jnp.float32(7.0376836292e-2)
    y = y * f - 1.1514610310e-1
    y = y * f + 1.1676998740e-1
    y = y * f - 1.2420140846e-1
    y = y * f + 1.4249322787e-1
    y = y * f - 1.6668057665e-1
    y = y * f + 2.0000714765e-1
    y = y * f - 2.4999993993e-1
    y = y * f + 3.3333331174e-1
    y = f * z * y
    y = y + ef * _LN2_LO
    y = y - 0.5 * z
    return f + y + ef * _LN2_HI


_mesh = plsc.VectorSubcoreMesh(core_axis_name="c", subcore_axis_name="s")


@functools.partial(
    pl.kernel,
    out_type=jax.ShapeDtypeStruct((T * LANES,), jnp.float32),
    mesh=_mesh,
    compiler_params=pltpu.CompilerParams(needs_layout_passes=False),
    scratch_types=[
        pltpu.VMEM_SHARED((NPAD + NDUMP,), jnp.float32),
        pltpu.VMEM((CHUNK,), jnp.float32),
        pltpu.VMEM((CHUNK,), jnp.float32),
        pltpu.VMEM((CHUNK,), jnp.float32),
        pltpu.VMEM((CHUNK,), jnp.float32),
        pltpu.VMEM((CHUNK,), jnp.float32),
        pltpu.VMEM((CHUNK,), jnp.float32),
        pltpu.VMEM((CHUNK,), jnp.float32),
        pltpu.VMEM((CHUNK,), jnp.float32),
        pltpu.VMEM((KROWS, IDXW), jnp.int32),
        pltpu.VMEM((T * LANES,), jnp.float32),
        pltpu.SemaphoreType.DMA,
    ],
)
def _routing_kernel(
    lat_hbm, logn_hbm, len_hbm, slope_hbm, dst_hbm, out_hbm,
    shared_up, q_v, in_v, up_v, lat_v, celc_v, xc_v, len_v, zero_v,
    idx_v, out_v, scat_sem,
):
    cid = lax.axis_index("c")
    sid = lax.axis_index("s")
    base = sid * CHUNK

    pltpu.sync_copy(dst_hbm.at[sid], idx_v)
    pltpu.sync_copy(len_hbm.at[pl.ds(base, CHUNK)], len_v)
    pltpu.sync_copy(logn_hbm.at[pl.ds(base, CHUNK)], up_v)
    pltpu.sync_copy(slope_hbm.at[pl.ds(base, CHUNK)], lat_v)

    @plsc.parallel_loop(0, CHUNK, step=LANES)
    def _init(off):
        ds = pl.ds(off, LANES)
        ln_n = up_v[ds]
        sl = lat_v[ds]
        ln = len_v[ds]
        celc_v[ds] = (5.0 / 3.0) * _C27 * jnp.exp(0.5 * _ln16(sl) - ln_n)
        xc_v[ds] = 1.0 / (14.4 * sl * ln)
        ones = jnp.full((LANES,), 1.0, jnp.float32)
        q_v[ds] = ones
        in_v[ds] = ones
        zero_v[ds] = jnp.zeros((LANES,), jnp.float32)

    pltpu.sync_copy(zero_v, shared_up.at[pl.ds(base, CHUNK)])
    plsc.subcore_barrier()

    def substep(s, _):
        descs = [
            pltpu.async_copy(
                q_v.at[pl.ds(k * IDXW, IDXW)],
                shared_up.at[idx_v.at[k]],
                scat_sem,
                add=True,
            )
            for k in range(KROWS)
        ]
        for d in descs:
            d.wait()
        plsc.subcore_barrier()

        pltpu.sync_copy(shared_up.at[pl.ds(base, CHUNK)], up_v)
        pltpu.sync_copy(zero_v, shared_up.at[pl.ds(base, CHUNK)])

        @plsc.parallel_loop(0, CHUNK, step=LANES, unroll=2)
        def _mc(off):
            ds = pl.ds(off, LANES)
            q = q_v[ds]
            inflow = lat_v[ds] + up_v[ds]
            qref = jnp.maximum(0.5 * (inflow + q), 1e-6)
            p01 = jnp.exp(0.1 * _ln16(qref))
            p02 = p01 * p01
            p04 = p02 * p02
            sqrtq = p04 * p01
            cel = jnp.maximum(celc_v[ds] * p02, 1e-4)
            rcel = 1.0 / cel
            k2 = 2.0 * len_v[ds] * rcel
            x = jnp.clip(0.5 - xc_v[ds] * sqrtq * rcel, 0.0, 0.5)
            a = k2 * x
            b = k2 - a
            rden = 1.0 / (b + DT_SUB)
            c0 = (DT_SUB - a) * rden
            c1 = (DT_SUB + a) * rden
            c2 = (b - DT_SUB) * rden
            q_v[ds] = jnp.maximum(c0 * inflow + c1 * in_v[ds] + c2 * q, 0.0)
            in_v[ds] = inflow

        plsc.subcore_barrier()
        return 0

    def timestep(t, _):
        pltpu.sync_copy(lat_hbm.at[t, pl.ds(base, CHUNK)], lat_v)
        lax.fori_loop(0, NSUB, substep, 0)
        out_v[pl.ds(t * LANES, LANES)] = q_v[pl.ds(OUT_VREG * LANES, LANES)]
        return 0

    lax.fori_loop(0, T, timestep, 0)

    @pl.when(jnp.logical_and(cid == 0, sid == OUT_SUBCORE))
    def _():
        pltpu.sync_copy(out_v, out_hbm)


def kernel(lateral_inflow, log_manning_n, length, slope, downstream_idx):
    pad = NPAD - N
    lat = jnp.pad(lateral_inflow, ((0, 0), (0, pad)))
    logn = jnp.pad(log_manning_n, (0, pad))
    leng = jnp.pad(length, (0, pad), constant_values=1000.0)
    slp = jnp.pad(slope, (0, pad), constant_values=0.01)
    pad_idx = NPAD + (jnp.arange(pad, dtype=jnp.int32) % NDUMP)
    dst = jnp.concatenate([downstream_idx.astype(jnp.int32), pad_idx])
    dst = dst.reshape(NS, KROWS, IDXW)
    out = _routing_kernel(lat, logn, leng, slp, dst)
    return out.reshape(T, LANES)[:, OUT_LANE]
